# baseline (speedup 1.0000x reference)
import functools

import jax
import jax.numpy as jnp
from jax import lax
from jax.experimental import pallas as pl
from jax.experimental.pallas import tpu as pltpu
from jax.experimental.pallas import tpu_sc as plsc

LEN = 128
NW = 32
KSLICES = 4


def _gather_hslice(x, h0, hn, tag):
    T, C, H, W = x.shape
    chunks = LEN * C
    per_w = chunks // NW
    a, b = 2 * (T - 1), 2 * (LEN - 1)

    mesh = plsc.VectorSubcoreMesh(core_axis_name="c", subcore_axis_name="s")

    @functools.partial(
        pl.kernel,
        out_type=jax.ShapeDtypeStruct((LEN, C, hn, W), x.dtype),
        mesh=mesh,
        scratch_types=[
            pltpu.VMEM((2, hn, W), x.dtype),
            pltpu.SemaphoreType.DMA,
            pltpu.SemaphoreType.DMA,
            pltpu.SemaphoreType.DMA,
            pltpu.SemaphoreType.DMA,
        ],
        name=f"sc_gather_{tag}",
    )
    def k(x_hbm, out_hbm, buf, si0, si1, so0, so1):
        wid = lax.axis_index("s") * 2 + lax.axis_index("c")
        base = wid * per_w
        sin = (si0, si1)
        sout = (so0, so1)

        def start_in(q, slot):
            o = base + q
            frame = o // C
            ch = o % C
            src = (frame * a + (LEN - 1)) // b
            return pltpu.async_copy(
                x_hbm.at[src, ch, pl.ds(h0, hn)], buf.at[slot], sin[slot]
            )

        def start_out(q, slot):
            o = base + q
            return pltpu.async_copy(
                buf.at[slot], out_hbm.at[o // C, o % C], sout[slot]
            )

        in_cp = [None, None]
        out_cp = [None, None]
        in_cp[0] = start_in(0, 0)
        for q in range(per_w):
            slot = q % 2
            nxt = (q + 1) % 2
            if q + 1 < per_w:
                if q >= 1:
                    out_cp[nxt].wait()
                in_cp[nxt] = start_in(q + 1, nxt)
            in_cp[slot].wait()
            out_cp[slot] = start_out(q, slot)
        out_cp[0].wait()
        out_cp[1].wait()

    return k(x)


def kernel(x):
    T, C, H, W = x.shape
    hn = H // KSLICES
    parts = [_gather_hslice(x, k * hn, hn, f"h{k}") for k in range(KSLICES)]
    acc = jnp.zeros((C, H, W, LEN), x.dtype)
    for k in range(KSLICES):
        zk = jnp.transpose(parts[k], (1, 2, 3, 0))
        acc = lax.dynamic_update_slice(acc, zk, (0, k * hn, 0, 0))
    return jnp.transpose(acc, (3, 0, 1, 2))


# speedup vs baseline: 1.4119x; 1.4119x over previous
import functools

import jax
import jax.numpy as jnp
from jax import lax
from jax.experimental import pallas as pl
from jax.experimental.pallas import tpu as pltpu
from jax.experimental.pallas import tpu_sc as plsc

LEN = 128
NW = 32
NBUF = 4
HSPLIT = 2


def kernel(x):
    T, C, H, W = x.shape
    hn = H // HSPLIT
    chunks = LEN * C * HSPLIT
    per_w = chunks // NW
    a, b = 2 * (T - 1), 2 * (LEN - 1)

    mesh = plsc.VectorSubcoreMesh(core_axis_name="c", subcore_axis_name="s")

    @functools.partial(
        pl.kernel,
        out_type=jax.ShapeDtypeStruct((LEN, C, H, W), x.dtype),
        mesh=mesh,
        scratch_types=[
            pltpu.VMEM((NBUF, hn, W), x.dtype),
            pltpu.SemaphoreType.DMA,
            pltpu.SemaphoreType.DMA,
            pltpu.SemaphoreType.DMA,
            pltpu.SemaphoreType.DMA,
            pltpu.SemaphoreType.DMA,
            pltpu.SemaphoreType.DMA,
            pltpu.SemaphoreType.DMA,
            pltpu.SemaphoreType.DMA,
        ],
    )
    def k(x_hbm, out_hbm, buf, *sems):
        sin = sems[:NBUF]
        sout = sems[NBUF:]
        wid = lax.axis_index("s") * 2 + lax.axis_index("c")
        base = wid * per_w

        def addr(q):
            o = base + q
            oc = o // HSPLIT
            hh = (o % HSPLIT) * hn
            frame = oc // C
            ch = oc % C
            return frame, ch, hh

        def start_in(q, slot):
            frame, ch, hh = addr(q)
            src = (frame * a + (LEN - 1)) // b
            return pltpu.async_copy(
                x_hbm.at[src, ch, pl.ds(hh, hn)], buf.at[slot], sin[slot]
            )

        def start_out(q, slot):
            frame, ch, hh = addr(q)
            return pltpu.async_copy(
                buf.at[slot], out_hbm.at[frame, ch, pl.ds(hh, hn)], sout[slot]
            )

        in_cp = [None] * NBUF
        out_cp = [None] * NBUF
        for s in range(NBUF - 1):
            in_cp[s] = start_in(s, s)
        for q in range(per_w):
            slot = q % NBUF
            nq = q + NBUF - 1
            if nq < per_w:
                nslot = nq % NBUF
                if out_cp[nslot] is not None:
                    out_cp[nslot].wait()
                in_cp[nslot] = start_in(nq, nslot)
            in_cp[slot].wait()
            out_cp[slot] = start_out(q, slot)
        for q in range(max(0, per_w - NBUF), per_w):
            out_cp[q % NBUF].wait()

    return k(x)


# speedup vs baseline: 1.4222x; 1.0073x over previous
import functools

import jax
import jax.numpy as jnp
from jax import lax
from jax.experimental import pallas as pl
from jax.experimental.pallas import tpu as pltpu
from jax.experimental.pallas import tpu_sc as plsc

LEN = 128
NW = 32


def kernel(x):
    T, C, H, W = x.shape
    chunks = LEN * C
    per_w = chunks // NW
    a, b = 2 * (T - 1), 2 * (LEN - 1)

    mesh = plsc.VectorSubcoreMesh(core_axis_name="c", subcore_axis_name="s")

    @functools.partial(
        pl.kernel,
        out_type=jax.ShapeDtypeStruct((LEN, C, H, W), x.dtype),
        mesh=mesh,
        scratch_types=[
            pltpu.VMEM((2, H, W), x.dtype),
            pltpu.SemaphoreType.DMA,
            pltpu.SemaphoreType.DMA,
            pltpu.SemaphoreType.DMA,
            pltpu.SemaphoreType.DMA,
        ],
    )
    def k(x_hbm, out_hbm, buf, si0, si1, so0, so1):
        wid = lax.axis_index("s") * 2 + lax.axis_index("c")
        base = wid * per_w
        sin = (si0, si1)
        sout = (so0, so1)

        def start_in(q, slot):
            o = base + q
            frame = o // C
            ch = o % C
            src = (frame * a + (LEN - 1)) // b
            return pltpu.async_copy(x_hbm.at[src, ch], buf.at[slot], sin[slot])

        def start_out(q, slot):
            o = base + q
            return pltpu.async_copy(buf.at[slot], out_hbm.at[o // C, o % C], sout[slot])

        in_cp = [None, None]
        out_cp = [None, None]
        in_cp[0] = start_in(0, 0)
        for q in range(per_w):
            slot = q % 2
            nxt = (q + 1) % 2
            if q + 1 < per_w:
                if q >= 1:
                    out_cp[nxt].wait()
                in_cp[nxt] = start_in(q + 1, nxt)
            in_cp[slot].wait()
            out_cp[slot] = start_out(q, slot)
        out_cp[0].wait()
        out_cp[1].wait()

    return k(x)
